# baseline (speedup 1.0000x reference)
import jax
import jax.numpy as jnp
from jax.experimental import pallas as pl
from jax.experimental.pallas import tpu as pltpu

_P = 4
_H = 96
_W = 96
_C = 96
_B = 32
_WO = _W + 2


def _out_copies(o_ref, mid_scr, rows_scr, sem_mid, sem_rows, slot, n):
    mid = pltpu.make_async_copy(
        mid_scr.at[slot], o_ref.at[n, :, pl.ds(1, _H), :], sem_mid.at[slot])
    top = pltpu.make_async_copy(
        rows_scr.at[slot, :, pl.ds(0, 1), :], o_ref.at[n, :, pl.ds(0, 1), :],
        sem_rows.at[slot, 0])
    bot = pltpu.make_async_copy(
        rows_scr.at[slot, :, pl.ds(1, 1), :], o_ref.at[n, :, pl.ds(_H + 1, 1), :],
        sem_rows.at[slot, 1])
    return mid, top, bot


def _body(x_ref, o_ref, mid_scr, rows_scr, sem_mid, sem_rows):
    n = pl.program_id(0)
    slot = jax.lax.rem(n, 2)
    m = jax.lax.rem(n, _P * _P)
    py = jax.lax.div(m, _P)
    px = jax.lax.rem(m, _P)
    is_top = py == 0
    is_bot = py == _P - 1
    is_left = px == 0
    is_right = px == _P - 1

    @pl.when(n >= 2)
    def _wait_prev():
        for cp in _out_copies(o_ref, mid_scr, rows_scr, sem_mid, sem_rows,
                              slot, n - 2):
            cp.wait()

    xb = x_ref[0]
    third = jnp.float32(1.0 / 3.0)

    left = (xb[:, :, 0] + xb[:, :, 1] + xb[:, :, 2]) * third
    right = (xb[:, :, _W - 3] + xb[:, :, _W - 2] + xb[:, :, _W - 1]) * third
    left = jnp.where(is_left, 0.0, left)
    right = jnp.where(is_right, 0.0, right)
    mid_scr[slot, :, :, 0:1] = left[:, :, None]
    mid_scr[slot, :, :, 1:_W + 1] = xb
    mid_scr[slot, :, :, _W + 1:_WO] = right[:, :, None]

    r_first = xb[:, 0, :]
    r_last = xb[:, _H - 1, :]

    def pad_row(r):
        z = jnp.zeros((r.shape[0], 2), r.dtype)
        rp = jnp.concatenate([r, z], axis=1)
        mid3 = (rp[:, 0:_W] + rp[:, 1:_W + 1] + rp[:, 2:_W + 2]) * third
        return jnp.concatenate([r[:, :1], mid3, r[:, _W - 1:]], axis=1)

    cidx = jax.lax.broadcasted_iota(jnp.int32, (1, _WO), 1)
    col_zero = (is_left & (cidx == 0)) | (is_right & (cidx == _WO - 1))
    row_top = jnp.where(col_zero | is_top, 0.0, pad_row(r_first))
    row_bot = jnp.where(col_zero | is_bot, 0.0, pad_row(r_last))
    rows_scr[slot, :, 0:1, :] = row_top[:, None, :]
    rows_scr[slot, :, 1:2, :] = row_bot[:, None, :]

    for cp in _out_copies(o_ref, mid_scr, rows_scr, sem_mid, sem_rows, slot, n):
        cp.start()

    @pl.when(n == _B - 1)
    def _drain():
        for cp in _out_copies(o_ref, mid_scr, rows_scr, sem_mid, sem_rows,
                              1 - slot, n - 1):
            cp.wait()
        for cp in _out_copies(o_ref, mid_scr, rows_scr, sem_mid, sem_rows,
                              slot, n):
            cp.wait()


def kernel(x):
    return pl.pallas_call(
        _body,
        grid=(_B,),
        in_specs=[pl.BlockSpec((1, _C, _H, _W), lambda n: (n, 0, 0, 0))],
        out_specs=pl.BlockSpec(memory_space=pltpu.MemorySpace.HBM),
        out_shape=jax.ShapeDtypeStruct((_B, _C, _H + 2, _WO), jnp.float32),
        scratch_shapes=[
            pltpu.VMEM((2, _C, _H, _WO), jnp.float32),
            pltpu.VMEM((2, _C, 2, _WO), jnp.float32),
            pltpu.SemaphoreType.DMA((2,)),
            pltpu.SemaphoreType.DMA((2, 2)),
        ],
        compiler_params=pltpu.CompilerParams(
            dimension_semantics=("arbitrary",),
        ),
    )(x)


# speedup vs baseline: 1.3773x; 1.3773x over previous
import jax
import jax.numpy as jnp
import numpy as np
from jax.experimental import pallas as pl
from jax.experimental.pallas import tpu as pltpu

_P = 4
_H = 96
_W = 96
_C = 96
_B = 32
_WO = _W + 2


def _mid_matrices():
    base = np.zeros((_W, _WO), np.float32)
    base[np.arange(_W), np.arange(_W) + 1] = 1.0
    base[0:3, 0] = 1.0 / 3.0
    base[_W - 3:_W, _WO - 1] = 1.0 / 3.0
    left0 = base.copy()
    left0[:, 0] = 0.0
    right0 = base.copy()
    right0[:, _WO - 1] = 0.0
    return np.stack([base, left0, right0])


def _row_matrix():
    m = np.zeros((_W, _WO), np.float32)
    for d in range(3):
        for j in range(1, _W + 1):
            w = j - 1 + d
            if w < _W:
                m[w, j] = 1.0 / 3.0
    m[0, 0] = 1.0
    m[_W - 1, _WO - 1] = 1.0
    return m


_MID_M = _mid_matrices()
_ROW_M = _row_matrix()


def _out_copies(o_ref, mid_scr, rows_scr, sem_mid, sem_rows, slot, n):
    mid = pltpu.make_async_copy(
        mid_scr.at[slot], o_ref.at[n, :, pl.ds(1, _H), :], sem_mid.at[slot])
    top = pltpu.make_async_copy(
        rows_scr.at[slot, :, pl.ds(0, 1), :], o_ref.at[n, :, pl.ds(0, 1), :],
        sem_rows.at[slot, 0])
    bot = pltpu.make_async_copy(
        rows_scr.at[slot, :, pl.ds(1, 1), :], o_ref.at[n, :, pl.ds(_H + 1, 1), :],
        sem_rows.at[slot, 1])
    return mid, top, bot


def _body(x_ref, m_ref, mrow_ref, o_ref, mid_scr, rows_scr, sem_mid, sem_rows):
    n = pl.program_id(0)
    slot = jax.lax.rem(n, 2)
    m = jax.lax.rem(n, _P * _P)
    py = jax.lax.div(m, _P)
    px = jax.lax.rem(m, _P)
    is_top = py == 0
    is_bot = py == _P - 1
    is_left = px == 0
    is_right = px == _P - 1

    @pl.when(n >= 2)
    def _wait_prev():
        for cp in _out_copies(o_ref, mid_scr, rows_scr, sem_mid, sem_rows,
                              slot, n - 2):
            cp.wait()

    xb = x_ref[0]

    x2d = xb.reshape(_C * _H, _W)
    mm = jax.lax.dot_general(x2d, m_ref[0], (((1,), (0,)), ((), ())),
                             preferred_element_type=jnp.float32)
    mid_scr[slot] = mm.reshape(_C, _H, _WO)

    rr = jnp.concatenate([xb[:, 0, :], xb[:, _H - 1, :]], axis=0)
    rmm = jax.lax.dot_general(rr, mrow_ref[...], (((1,), (0,)), ((), ())),
                              preferred_element_type=jnp.float32)
    cidx = jax.lax.broadcasted_iota(jnp.int32, (1, _WO), 1)
    col_zero = (is_left & (cidx == 0)) | (is_right & (cidx == _WO - 1))
    row_top = jnp.where(col_zero | is_top, 0.0, rmm[:_C])
    row_bot = jnp.where(col_zero | is_bot, 0.0, rmm[_C:])
    rows_scr[slot, :, 0:1, :] = row_top[:, None, :]
    rows_scr[slot, :, 1:2, :] = row_bot[:, None, :]

    for cp in _out_copies(o_ref, mid_scr, rows_scr, sem_mid, sem_rows, slot, n):
        cp.start()

    @pl.when(n == _B - 1)
    def _drain():
        for cp in _out_copies(o_ref, mid_scr, rows_scr, sem_mid, sem_rows,
                              1 - slot, n - 1):
            cp.wait()
        for cp in _out_copies(o_ref, mid_scr, rows_scr, sem_mid, sem_rows,
                              slot, n):
            cp.wait()


def _m_index(n):
    px = jax.lax.rem(n, _P)
    return (jnp.where(px == 0, 1, jnp.where(px == _P - 1, 2, 0)), 0, 0)


def kernel(x):
    return pl.pallas_call(
        _body,
        grid=(_B,),
        in_specs=[
            pl.BlockSpec((1, _C, _H, _W), lambda n: (n, 0, 0, 0)),
            pl.BlockSpec((1, _W, _WO), _m_index),
            pl.BlockSpec((_W, _WO), lambda n: (0, 0)),
        ],
        out_specs=pl.BlockSpec(memory_space=pltpu.MemorySpace.HBM),
        out_shape=jax.ShapeDtypeStruct((_B, _C, _H + 2, _WO), jnp.float32),
        scratch_shapes=[
            pltpu.VMEM((2, _C, _H, _WO), jnp.float32),
            pltpu.VMEM((2, _C, 2, _WO), jnp.float32),
            pltpu.SemaphoreType.DMA((2,)),
            pltpu.SemaphoreType.DMA((2, 2)),
        ],
        compiler_params=pltpu.CompilerParams(
            dimension_semantics=("arbitrary",),
        ),
    )(x, jnp.asarray(_MID_M), jnp.asarray(_ROW_M))
